# baseline (speedup 1.0000x reference)
import jax
import jax.numpy as jnp
from jax.experimental import pallas as pl

_BM = 512


def _fused_kernel(x_ref, w_ref, c_ref, o_ref):
    s = jax.lax.dot_general(
        x_ref[...], w_ref[...],
        dimension_numbers=(((1,), (1,)), ((), ())),
        preferred_element_type=jnp.float32,
    )
    mn = jnp.min(s, axis=1, keepdims=True) - 1.0
    o_ref[...] = jnp.where(c_ref[...] == 0, mn, s)


def kernel(X, wK, cK):
    M, K = X.shape
    N = wK.shape[0]
    c2d = cK.reshape(1, N)
    grid = (M // _BM,)
    return pl.pallas_call(
        _fused_kernel,
        grid=grid,
        in_specs=[
            pl.BlockSpec((_BM, K), lambda i: (i, 0)),
            pl.BlockSpec((N, K), lambda i: (0, 0)),
            pl.BlockSpec((1, N), lambda i: (0, 0)),
        ],
        out_specs=pl.BlockSpec((_BM, N), lambda i: (i, 0)),
        out_shape=jax.ShapeDtypeStruct((M, N), jnp.float32),
    )(X, wK, c2d)
